# baseline (speedup 1.0000x reference)
import functools

import jax
import jax.numpy as jnp
from jax import lax
from jax.experimental import pallas as pl
from jax.experimental.pallas import tpu as pltpu
from jax.experimental.pallas import tpu_sc as plsc

_NC = 2
_NS = 16
_NW = _NC * _NS
_LANES = 16
_CHUNK = 16384


def kernel(x, pos_emb_table):
    B, S, H = x.shape
    n = B * S * H
    tsz = S * H
    per_w = n // _NW
    nchunks = per_w // _CHUNK
    xf = x.reshape(n)
    tf = pos_emb_table[:S].reshape(tsz)

    mesh = plsc.VectorSubcoreMesh(
        core_axis_name="c", subcore_axis_name="s",
        num_cores=_NC, num_subcores=_NS,
    )

    @functools.partial(
        pl.kernel,
        mesh=mesh,
        out_type=jax.ShapeDtypeStruct((n,), jnp.float32),
        scratch_types=[
            pltpu.VMEM((_CHUNK,), jnp.float32),
            pltpu.VMEM((_CHUNK,), jnp.float32),
        ],
    )
    def sc_add(x_hbm, t_hbm, o_hbm, xv, tv):
        wid = lax.axis_index("s") * _NC + lax.axis_index("c")
        base = wid * per_w

        def chunk_body(ci, carry):
            off = base + ci * _CHUNK
            toff = lax.rem(off, tsz)
            pltpu.sync_copy(x_hbm.at[pl.ds(off, _CHUNK)], xv)
            pltpu.sync_copy(t_hbm.at[pl.ds(toff, _CHUNK)], tv)

            def vec_body(j, c):
                sl = pl.ds(j * _LANES, _LANES)
                xv[sl] = xv[sl] + tv[sl]
                return c

            lax.fori_loop(0, _CHUNK // _LANES, vec_body, 0)
            pltpu.sync_copy(xv, o_hbm.at[pl.ds(off, _CHUNK)])
            return carry

        lax.fori_loop(0, nchunks, chunk_body, 0)

    out = sc_add(xf, tf)
    return out.reshape(B, S, H)


# speedup vs baseline: 8.5896x; 8.5896x over previous
import jax
import jax.numpy as jnp
from jax.experimental import pallas as pl


_S_BLK = 2048


def _add_kernel(x_ref, t_ref, o_ref):
    o_ref[0] = x_ref[0] + t_ref[...]


def kernel(x, pos_emb_table):
    B, S, H = x.shape
    table = pos_emb_table[:S]
    grid = (S // _S_BLK, B)
    return pl.pallas_call(
        _add_kernel,
        grid=grid,
        in_specs=[
            pl.BlockSpec((1, _S_BLK, H), lambda s, b: (b, s, 0)),
            pl.BlockSpec((_S_BLK, H), lambda s, b: (s, 0)),
        ],
        out_specs=pl.BlockSpec((1, _S_BLK, H), lambda s, b: (b, s, 0)),
        out_shape=jax.ShapeDtypeStruct((B, S, H), x.dtype),
    )(x, table)
